# baseline (speedup 1.0000x reference)
import functools

import jax
import jax.numpy as jnp
from jax import lax
from jax.experimental import pallas as pl
from jax.experimental.pallas import tpu as pltpu
from jax.experimental.pallas import tpu_sc as plsc

B, C, H, W = 16, 34, 128, 128
K = 32
HW = H * W
L = 16
NPT = K * C
CHUNK = 128
NCH = NPT // CHUNK
REM = NPT - NCH * CHUNK


def _body(out_hbm, ind_hbm, m_hbm, t_hbm, res_hbm,
          ind_v, idx_v, pred_v, m_v, t_v, part_v, all_v, out_v, part_hbm,
          sem, sem2):
    b = lax.axis_index("s")

    pltpu.sync_copy(ind_hbm.at[b], ind_v)
    mcp = pltpu.async_copy(m_hbm.at[b], m_v, sem2)
    tcp = pltpu.async_copy(t_hbm.at[b], t_v, sem2)

    base_b = b * (C * HW)
    ind0 = ind_v[pl.ds(0, L)]
    ind1 = ind_v[pl.ds(L, L)]

    def build(c, _):
        off = base_b + c * HW
        idx_v[pl.ds(c * L, L)] = ind0 + off
        idx_v[pl.ds((C + c) * L, L)] = ind1 + off
        return 0

    lax.fori_loop(0, C, build, 0)

    gcp = pltpu.async_copy(out_hbm.at[idx_v], pred_v, sem)
    mcp.wait()
    tcp.wait()
    gcp.wait()

    lane = jnp.arange(L, dtype=jnp.int32)
    cvec = jnp.zeros((L,), jnp.int32)

    def step(c, carry):
        acc, mac = carry
        p0 = pred_v[pl.ds(c * L, L)]
        m0 = plsc.load_gather(m_v, [lane, cvec + c])
        t0 = plsc.load_gather(t_v, [lane, cvec + c])
        acc = acc + jnp.abs(m0 * (p0 - t0))
        mac = mac + m0
        p1 = pred_v[pl.ds((C + c) * L, L)]
        m1 = plsc.load_gather(m_v, [lane + L, cvec + c])
        t1 = plsc.load_gather(t_v, [lane + L, cvec + c])
        acc = acc + jnp.abs(m1 * (p1 - t1))
        mac = mac + m1
        return acc, mac

    acc, mac = lax.fori_loop(
        0, C, step,
        (jnp.zeros((L,), jnp.float32), jnp.zeros((L,), jnp.float32)))
    part_v[pl.ds(0, L)] = acc
    part_v[pl.ds(L, L)] = mac

    pltpu.sync_copy(part_v, part_hbm.at[b])
    plsc.subcore_barrier()

    @pl.when(b == 0)
    def _():
        pltpu.sync_copy(part_hbm, all_v)

        def red(i, carry):
            a, m = carry
            return a + all_v[i, pl.ds(0, L)], m + all_v[i, pl.ds(L, L)]

        a, m = lax.fori_loop(
            0, B, red,
            (jnp.zeros((L,), jnp.float32), jnp.zeros((L,), jnp.float32)))
        num = jnp.full((L,), jnp.sum(a), jnp.float32)
        den = jnp.full((L,), jnp.sum(m), jnp.float32) + 0.0001
        out_v[...] = num / den
        pltpu.sync_copy(out_v.at[pl.ds(0, 1)], res_hbm)


@jax.jit
def kernel(output, mask, ind, target):
    out_flat = output.reshape(-1)
    ind32 = ind.astype(jnp.int32)

    mesh = plsc.VectorSubcoreMesh(
        core_axis_name="c", subcore_axis_name="s", num_cores=1)
    run = functools.partial(
        pl.kernel,
        out_type=jax.ShapeDtypeStruct((1,), jnp.float32),
        mesh=mesh,
        scratch_types=[
            pltpu.VMEM((K,), jnp.int32),
            pltpu.VMEM((NPT,), jnp.int32),
            pltpu.VMEM((NPT,), jnp.float32),
            pltpu.VMEM((K, C), jnp.float32),
            pltpu.VMEM((K, C), jnp.float32),
            pltpu.VMEM((2 * L,), jnp.float32),
            pltpu.VMEM((B, 2 * L), jnp.float32),
            pltpu.VMEM((L,), jnp.float32),
            pltpu.HBM((B, 2 * L), jnp.float32),
            pltpu.SemaphoreType.DMA,
            pltpu.SemaphoreType.DMA,
        ],
        compiler_params=pltpu.CompilerParams(needs_layout_passes=False),
    )(_body)
    res = run(out_flat, ind32, mask, target)
    return res.reshape(())


# speedup vs baseline: 1.0030x; 1.0030x over previous
import functools

import jax
import jax.numpy as jnp
from jax import lax
from jax.experimental import pallas as pl
from jax.experimental.pallas import tpu as pltpu
from jax.experimental.pallas import tpu_sc as plsc

B, C, H, W = 16, 34, 128, 128
K = 32
HW = H * W
L = 16
NPT = K * C
CHUNK = 128
NCH = NPT // CHUNK
REM = NPT - NCH * CHUNK


def _body(out_hbm, ind_hbm, m_hbm, t_hbm, res_hbm,
          ind_v, idx_v, pred_v, m_v, t_v, part_v, all_v, out_v, part_hbm,
          sem, sem2):
    b = lax.axis_index("s")

    pltpu.sync_copy(ind_hbm.at[b], ind_v)
    mcp = pltpu.async_copy(m_hbm.at[b], m_v, sem2)
    tcp = pltpu.async_copy(t_hbm.at[b], t_v, sem2)

    base_b = b * (C * HW)
    ind0 = ind_v[pl.ds(0, L)]
    ind1 = ind_v[pl.ds(L, L)]

    def build(c, _):
        off = base_b + c * HW
        idx_v[pl.ds(c * L, L)] = ind0 + off
        idx_v[pl.ds((C + c) * L, L)] = ind1 + off
        return 0

    lax.fori_loop(0, C, build, 0)

    gcp = pltpu.async_copy(out_hbm.at[idx_v], pred_v, sem)
    mcp.wait()
    tcp.wait()
    gcp.wait()

    lane = jnp.arange(L, dtype=jnp.int32)
    cvec = jnp.zeros((L,), jnp.int32)

    def step(c, carry):
        acc, mac = carry
        p0 = pred_v[pl.ds(c * L, L)]
        m0 = plsc.load_gather(m_v, [lane, cvec + c])
        t0 = plsc.load_gather(t_v, [lane, cvec + c])
        acc = acc + jnp.abs(m0 * (p0 - t0))
        mac = mac + m0
        p1 = pred_v[pl.ds((C + c) * L, L)]
        m1 = plsc.load_gather(m_v, [lane + L, cvec + c])
        t1 = plsc.load_gather(t_v, [lane + L, cvec + c])
        acc = acc + jnp.abs(m1 * (p1 - t1))
        mac = mac + m1
        return acc, mac

    acc, mac = lax.fori_loop(
        0, C, step,
        (jnp.zeros((L,), jnp.float32), jnp.zeros((L,), jnp.float32)))
    part_v[pl.ds(0, L)] = acc
    part_v[pl.ds(L, L)] = mac

    pltpu.sync_copy(part_v, part_hbm.at[b])
    plsc.subcore_barrier()

    @pl.when(b == 0)
    def _():
        pltpu.sync_copy(part_hbm, all_v)

        def red(i, carry):
            a, m = carry
            return a + all_v[i, pl.ds(0, L)], m + all_v[i, pl.ds(L, L)]

        a, m = lax.fori_loop(
            0, B, red,
            (jnp.zeros((L,), jnp.float32), jnp.zeros((L,), jnp.float32)))
        num = jnp.full((L,), jnp.sum(a), jnp.float32)
        den = jnp.full((L,), jnp.sum(m), jnp.float32) + 0.0001
        out_v[...] = num / den
        pltpu.sync_copy(out_v.at[pl.ds(0, 1)], res_hbm)


@jax.jit
def kernel(output, mask, ind, target):
    out_flat = output.reshape(-1)
    ind32 = ind.astype(jnp.int32)

    mesh = plsc.VectorSubcoreMesh(
        core_axis_name="c", subcore_axis_name="s", num_cores=1)
    run = functools.partial(
        pl.kernel,
        out_type=jax.ShapeDtypeStruct((1,), jnp.float32),
        mesh=mesh,
        scratch_types=[
            pltpu.VMEM((K,), jnp.int32),
            pltpu.VMEM((NPT,), jnp.int32),
            pltpu.VMEM((NPT,), jnp.float32),
            pltpu.VMEM((K, C), jnp.float32),
            pltpu.VMEM((K, C), jnp.float32),
            pltpu.VMEM((2 * L,), jnp.float32),
            pltpu.VMEM((B, 2 * L), jnp.float32),
            pltpu.VMEM((L,), jnp.float32),
            pltpu.HBM((B, 2 * L), jnp.float32),
            pltpu.SemaphoreType.DMA,
            pltpu.SemaphoreType.DMA,
        ],
        compiler_params=pltpu.CompilerParams(
            needs_layout_passes=False,
            disable_bounds_checks=True,
            disable_semaphore_checks=True,
        ),
    )(_body)
    res = run(out_flat, ind32, mask, target)
    return res.reshape(())
